# Initial kernel scaffold; baseline (speedup 1.0000x reference)
import jax
import jax.numpy as jnp
from jax.experimental import pallas as pl


def kernel(hidden_states, Wr, sg, su, sd, Wg, Wu, Wd):
    raise NotImplementedError("write your pallas kernel here")


# speedup vs baseline: 1.4147x; 1.4147x over previous
import jax
import jax.numpy as jnp
from jax.experimental import pallas as pl
from jax.experimental.pallas import tpu as pltpu

_BT = 256


def _silu(x):
    return x * jax.nn.sigmoid(x)


def _moe_body(x_ref, wr_ref, sg_ref, su_ref, sd_ref, wg_ref, wu_ref, wd_ref,
              out_ref, logits_ref, wall_ref):
    e = pl.program_id(1)
    xb = x_ref[...]

    @pl.when(e == 0)
    def _prologue():
        logits = jnp.dot(xb, wr_ref[...], preferred_element_type=jnp.float32)
        logits_ref[...] = logits
        probs = jax.nn.softmax(logits, axis=-1)
        eidx = jax.lax.broadcasted_iota(jnp.int32, probs.shape, 1)
        big = jnp.int32(probs.shape[1])
        m1 = jnp.max(probs, axis=-1, keepdims=True)
        i1 = jnp.min(jnp.where(probs == m1, eidx, big), axis=-1, keepdims=True)
        masked = jnp.where(eidx == i1, -jnp.inf, probs)
        m2 = jnp.max(masked, axis=-1, keepdims=True)
        i2 = jnp.min(jnp.where(masked == m2, eidx, big), axis=-1, keepdims=True)
        s = m1 + m2
        wall_ref[...] = (jnp.where(eidx == i1, m1 / s, 0.0)
                         + jnp.where(eidx == i2, m2 / s, 0.0))
        g = jnp.dot(xb, sg_ref[...], preferred_element_type=jnp.float32)
        u = jnp.dot(xb, su_ref[...], preferred_element_type=jnp.float32)
        out_ref[...] = jnp.dot(_silu(g) * u, sd_ref[...],
                               preferred_element_type=jnp.float32)

    g = jnp.dot(xb, wg_ref[0], preferred_element_type=jnp.float32)
    u = jnp.dot(xb, wu_ref[0], preferred_element_type=jnp.float32)
    y = jnp.dot(_silu(g) * u, wd_ref[0], preferred_element_type=jnp.float32)
    eidx = jax.lax.broadcasted_iota(jnp.int32, wall_ref.shape, 1)
    w_e = jnp.sum(jnp.where(eidx == e, wall_ref[...], 0.0), axis=-1,
                  keepdims=True)
    out_ref[...] += w_e * y


def kernel(hidden_states, Wr, sg, su, sd, Wg, Wu, Wd):
    b, s, h = hidden_states.shape
    t = b * s
    x = hidden_states.reshape(t, h)
    e_num = Wr.shape[1]
    f = Wg.shape[2]
    fs = sg.shape[1]

    grid = (t // _BT, e_num)
    out, logits = pl.pallas_call(
        _moe_body,
        grid=grid,
        in_specs=[
            pl.BlockSpec((_BT, h), lambda i, e: (i, 0)),
            pl.BlockSpec((h, e_num), lambda i, e: (0, 0)),
            pl.BlockSpec((h, fs), lambda i, e: (0, 0)),
            pl.BlockSpec((h, fs), lambda i, e: (0, 0)),
            pl.BlockSpec((fs, h), lambda i, e: (0, 0)),
            pl.BlockSpec((1, h, f), lambda i, e: (e, 0, 0)),
            pl.BlockSpec((1, h, f), lambda i, e: (e, 0, 0)),
            pl.BlockSpec((1, f, h), lambda i, e: (e, 0, 0)),
        ],
        out_specs=[
            pl.BlockSpec((_BT, h), lambda i, e: (i, 0)),
            pl.BlockSpec((_BT, e_num), lambda i, e: (i, 0)),
        ],
        out_shape=[
            jax.ShapeDtypeStruct((t, h), jnp.float32),
            jax.ShapeDtypeStruct((t, e_num), jnp.float32),
        ],
        scratch_shapes=[pltpu.VMEM((_BT, e_num), jnp.float32)],
        compiler_params=pltpu.CompilerParams(
            dimension_semantics=("parallel", "arbitrary"),
        ),
    )(x, Wr, sg, su, sd, Wg, Wu, Wd)
    return out.reshape(b, s, h), logits


# speedup vs baseline: 1.4727x; 1.0409x over previous
import jax
import jax.numpy as jnp
from jax.experimental import pallas as pl
from jax.experimental.pallas import tpu as pltpu

_BT = 256


def _silu(x):
    return x * jax.nn.sigmoid(x)


def _moe_body(x_ref, wr_ref, sg_ref, su_ref, sd_ref, wg_ref, wu_ref, wd_ref,
              out_ref, logits_ref, wall_ref):
    e = pl.program_id(1)
    xb = x_ref[...]

    xh = xb.astype(jnp.bfloat16)

    @pl.when(e == 0)
    def _prologue():
        logits = jnp.dot(xb, wr_ref[...], preferred_element_type=jnp.float32)
        logits_ref[...] = logits
        probs = jax.nn.softmax(logits, axis=-1)
        eidx = jax.lax.broadcasted_iota(jnp.int32, probs.shape, 1)
        big = jnp.int32(probs.shape[1])
        m1 = jnp.max(probs, axis=-1, keepdims=True)
        i1 = jnp.min(jnp.where(probs == m1, eidx, big), axis=-1, keepdims=True)
        masked = jnp.where(eidx == i1, -jnp.inf, probs)
        m2 = jnp.max(masked, axis=-1, keepdims=True)
        i2 = jnp.min(jnp.where(masked == m2, eidx, big), axis=-1, keepdims=True)
        s = m1 + m2
        wall_ref[...] = (jnp.where(eidx == i1, m1 / s, 0.0)
                         + jnp.where(eidx == i2, m2 / s, 0.0))
        g = jnp.dot(xh, sg_ref[...], preferred_element_type=jnp.float32)
        u = jnp.dot(xh, su_ref[...], preferred_element_type=jnp.float32)
        hidd = (_silu(g) * u).astype(jnp.bfloat16)
        out_ref[...] = jnp.dot(hidd, sd_ref[...],
                               preferred_element_type=jnp.float32)

    g = jnp.dot(xh, wg_ref[0], preferred_element_type=jnp.float32)
    u = jnp.dot(xh, wu_ref[0], preferred_element_type=jnp.float32)
    y = jnp.dot((_silu(g) * u).astype(jnp.bfloat16), wd_ref[0],
                preferred_element_type=jnp.float32)
    eidx = jax.lax.broadcasted_iota(jnp.int32, wall_ref.shape, 1)
    w_e = jnp.sum(jnp.where(eidx == e, wall_ref[...], 0.0), axis=-1,
                  keepdims=True)
    out_ref[...] += w_e * y


def kernel(hidden_states, Wr, sg, su, sd, Wg, Wu, Wd):
    b, s, h = hidden_states.shape
    t = b * s
    x = hidden_states.reshape(t, h)
    e_num = Wr.shape[1]
    f = Wg.shape[2]
    fs = sg.shape[1]

    sgh = sg.astype(jnp.bfloat16)
    suh = su.astype(jnp.bfloat16)
    sdh = sd.astype(jnp.bfloat16)
    Wgh = Wg.astype(jnp.bfloat16)
    Wuh = Wu.astype(jnp.bfloat16)
    Wdh = Wd.astype(jnp.bfloat16)

    grid = (t // _BT, e_num)
    out, logits = pl.pallas_call(
        _moe_body,
        grid=grid,
        in_specs=[
            pl.BlockSpec((_BT, h), lambda i, e: (i, 0)),
            pl.BlockSpec((h, e_num), lambda i, e: (0, 0)),
            pl.BlockSpec((h, fs), lambda i, e: (0, 0)),
            pl.BlockSpec((h, fs), lambda i, e: (0, 0)),
            pl.BlockSpec((fs, h), lambda i, e: (0, 0)),
            pl.BlockSpec((1, h, f), lambda i, e: (e, 0, 0)),
            pl.BlockSpec((1, h, f), lambda i, e: (e, 0, 0)),
            pl.BlockSpec((1, f, h), lambda i, e: (e, 0, 0)),
        ],
        out_specs=[
            pl.BlockSpec((_BT, h), lambda i, e: (i, 0)),
            pl.BlockSpec((_BT, e_num), lambda i, e: (i, 0)),
        ],
        out_shape=[
            jax.ShapeDtypeStruct((t, h), jnp.float32),
            jax.ShapeDtypeStruct((t, e_num), jnp.float32),
        ],
        scratch_shapes=[pltpu.VMEM((_BT, e_num), jnp.float32)],
        compiler_params=pltpu.CompilerParams(
            dimension_semantics=("parallel", "arbitrary"),
        ),
    )(x, Wr, sgh, suh, sdh, Wgh, Wuh, Wdh)
    return out.reshape(b, s, h), logits
